# baseline (speedup 1.0000x reference)
import functools

import jax
import jax.numpy as jnp
from jax.experimental import pallas as pl
from jax.experimental.pallas import tpu as pltpu

N = 10000
E = 320000
D = 128
ED = 16

BE = 2560
BN = 2000


def _silu(v):
    return v * jax.nn.sigmoid(v)


def _edge_body(hrow_ref, hcol_ref, radial_ref, ea_ref,
               We1a_ref, We1b_ref, we1r_ref, We1e_ref, be1_ref,
               We2_ref, be2_ref, Wc1_ref, bc1_ref, wc2_ref, wa_ref, ba_ref,
               m_ref, cw_ref):
    hrow = hrow_ref[...]
    hcol = hcol_ref[...]
    radial = radial_ref[...]
    ea = ea_ref[...]
    pre = (
        jnp.dot(hrow, We1a_ref[...], preferred_element_type=jnp.float32)
        + jnp.dot(hcol, We1b_ref[...], preferred_element_type=jnp.float32)
        + radial * we1r_ref[...]
        + jnp.dot(ea, We1e_ref[...], preferred_element_type=jnp.float32)
        + be1_ref[...]
    )
    m = _silu(pre)
    m = _silu(jnp.dot(m, We2_ref[...], preferred_element_type=jnp.float32)
              + be2_ref[...])
    att = jax.nn.sigmoid(
        jnp.sum(m * wa_ref[...], axis=1, keepdims=True) + ba_ref[0, 0])
    m = m * att
    t = _silu(jnp.dot(m, Wc1_ref[...], preferred_element_type=jnp.float32)
              + bc1_ref[...])
    cw = jnp.sum(t * wc2_ref[...], axis=1, keepdims=True)
    m_ref[...] = m
    cw_ref[...] = cw


def _node_body(h_ref, mi_ref, Wn1a_ref, Wn1b_ref, bn1_ref, Wn2_ref, bn2_ref,
               hout_ref):
    h = h_ref[...]
    mi = mi_ref[...]
    t = _silu(
        jnp.dot(h, Wn1a_ref[...], preferred_element_type=jnp.float32)
        + jnp.dot(mi, Wn1b_ref[...], preferred_element_type=jnp.float32)
        + bn1_ref[...]
    )
    hout_ref[...] = h + jnp.dot(t, Wn2_ref[...],
                                preferred_element_type=jnp.float32) + bn2_ref[...]


def _w_spec(shape):
    return pl.BlockSpec(shape, lambda i: (0,) * len(shape))


def kernel(h, x, edge_index, edge_attr, We1, be1, We2, be2, Wc1, bc1, Wc2,
           Wa, ba, Wn1, bn1, Wn2, bn2):
    row = edge_index[0].astype(jnp.int32)
    col = edge_index[1].astype(jnp.int32)

    hrow = h[row]
    hcol = h[col]
    coord_diff = x[row] - x[col]
    radial = jnp.sum(coord_diff ** 2, axis=-1, keepdims=True)

    We1a = We1[:D]
    We1b = We1[D:2 * D]
    we1r = We1[2 * D:2 * D + 1]
    We1e = We1[2 * D + 1:]

    grid_e = (E // BE,)
    m, cw = pl.pallas_call(
        _edge_body,
        grid=grid_e,
        in_specs=[
            pl.BlockSpec((BE, D), lambda i: (i, 0)),
            pl.BlockSpec((BE, D), lambda i: (i, 0)),
            pl.BlockSpec((BE, 1), lambda i: (i, 0)),
            pl.BlockSpec((BE, ED), lambda i: (i, 0)),
            _w_spec((D, D)), _w_spec((D, D)), _w_spec((1, D)),
            _w_spec((ED, D)), _w_spec((1, D)),
            _w_spec((D, D)), _w_spec((1, D)),
            _w_spec((D, D)), _w_spec((1, D)),
            _w_spec((1, D)), _w_spec((1, D)), _w_spec((1, 1)),
        ],
        out_specs=[
            pl.BlockSpec((BE, D), lambda i: (i, 0)),
            pl.BlockSpec((BE, 1), lambda i: (i, 0)),
        ],
        out_shape=[
            jax.ShapeDtypeStruct((E, D), jnp.float32),
            jax.ShapeDtypeStruct((E, 1), jnp.float32),
        ],
    )(hrow, hcol, radial, edge_attr,
      We1a, We1b, we1r, We1e, be1[None, :],
      We2, be2[None, :], Wc1, bc1[None, :], Wc2.T, Wa.T, ba[None, :])

    coord_update = coord_diff * (cw / jnp.sqrt(radial + 1e-08))
    x_out = x.at[row].add(coord_update)
    m_i = jnp.zeros((N, D), jnp.float32).at[row].add(m)

    Wn1a = Wn1[:D]
    Wn1b = Wn1[D:]
    grid_n = (N // BN,)
    h_out = pl.pallas_call(
        _node_body,
        grid=grid_n,
        in_specs=[
            pl.BlockSpec((BN, D), lambda i: (i, 0)),
            pl.BlockSpec((BN, D), lambda i: (i, 0)),
            _w_spec((D, D)), _w_spec((D, D)), _w_spec((1, D)),
            _w_spec((D, D)), _w_spec((1, D)),
        ],
        out_specs=pl.BlockSpec((BN, D), lambda i: (i, 0)),
        out_shape=jax.ShapeDtypeStruct((N, D), jnp.float32),
    )(h, m_i, Wn1a, Wn1b, bn1[None, :], Wn2, bn2[None, :])

    return (h_out, x_out)


# speedup vs baseline: 1.0442x; 1.0442x over previous
import jax
import jax.numpy as jnp
from jax import lax
from jax.experimental import pallas as pl

N = 10000
E = 320000
D = 128
ED = 16

BE = 2560
BN = 2000


def _silu(v):
    return v * jax.nn.sigmoid(v)


def _pre_body(h_ref, We1a_ref, We1b_ref, be1_ref, g1_ref, g2_ref):
    h = h_ref[...]
    g1_ref[...] = jnp.dot(h, We1a_ref[...],
                          preferred_element_type=jnp.float32) + be1_ref[...]
    g2_ref[...] = jnp.dot(h, We1b_ref[...], preferred_element_type=jnp.float32)


def _edge_body(g1r_ref, g2c_ref, radial_ref, ea_ref,
               we1r_ref, We1e_ref,
               We2_ref, be2_ref, Wc1_ref, bc1_ref, wc2_ref, wa_ref, ba_ref,
               m_ref, cw_ref):
    radial = radial_ref[...]
    pre = (
        g1r_ref[...] + g2c_ref[...]
        + radial * we1r_ref[...]
        + jnp.dot(ea_ref[...], We1e_ref[...], preferred_element_type=jnp.float32)
    )
    m = _silu(pre)
    m = _silu(jnp.dot(m, We2_ref[...], preferred_element_type=jnp.float32)
              + be2_ref[...])
    att = jax.nn.sigmoid(
        jnp.sum(m * wa_ref[...], axis=1, keepdims=True) + ba_ref[0, 0])
    m = m * att
    t = _silu(jnp.dot(m, Wc1_ref[...], preferred_element_type=jnp.float32)
              + bc1_ref[...])
    cw = jnp.sum(t * wc2_ref[...], axis=1, keepdims=True)
    m_ref[...] = m
    cw_ref[...] = cw * lax.rsqrt(radial + 1e-08)


def _node_body(h_ref, mi_ref, Wn1a_ref, Wn1b_ref, bn1_ref, Wn2_ref, bn2_ref,
               hout_ref):
    h = h_ref[...]
    t = _silu(
        jnp.dot(h, Wn1a_ref[...], preferred_element_type=jnp.float32)
        + jnp.dot(mi_ref[...], Wn1b_ref[...], preferred_element_type=jnp.float32)
        + bn1_ref[...]
    )
    hout_ref[...] = h + jnp.dot(t, Wn2_ref[...],
                                preferred_element_type=jnp.float32) + bn2_ref[...]


def _w_spec(shape):
    return pl.BlockSpec(shape, lambda i: (0,) * len(shape))


def kernel(h, x, edge_index, edge_attr, We1, be1, We2, be2, Wc1, bc1, Wc2,
           Wa, ba, Wn1, bn1, Wn2, bn2):
    row = edge_index[0].astype(jnp.int32)
    col = edge_index[1].astype(jnp.int32)

    We1a = We1[:D]
    We1b = We1[D:2 * D]
    we1r = We1[2 * D:2 * D + 1]
    We1e = We1[2 * D + 1:]

    g1, g2 = pl.pallas_call(
        _pre_body,
        grid=(N // BN,),
        in_specs=[
            pl.BlockSpec((BN, D), lambda i: (i, 0)),
            _w_spec((D, D)), _w_spec((D, D)), _w_spec((1, D)),
        ],
        out_specs=[
            pl.BlockSpec((BN, D), lambda i: (i, 0)),
            pl.BlockSpec((BN, D), lambda i: (i, 0)),
        ],
        out_shape=[
            jax.ShapeDtypeStruct((N, D), jnp.float32),
            jax.ShapeDtypeStruct((N, D), jnp.float32),
        ],
    )(h, We1a, We1b, be1[None, :])

    g1r = g1[row]
    g2c = g2[col]
    coord_diff = x[row] - x[col]
    radial = jnp.sum(coord_diff ** 2, axis=-1, keepdims=True)

    m, cwn = pl.pallas_call(
        _edge_body,
        grid=(E // BE,),
        in_specs=[
            pl.BlockSpec((BE, D), lambda i: (i, 0)),
            pl.BlockSpec((BE, D), lambda i: (i, 0)),
            pl.BlockSpec((BE, 1), lambda i: (i, 0)),
            pl.BlockSpec((BE, ED), lambda i: (i, 0)),
            _w_spec((1, D)), _w_spec((ED, D)),
            _w_spec((D, D)), _w_spec((1, D)),
            _w_spec((D, D)), _w_spec((1, D)),
            _w_spec((1, D)), _w_spec((1, D)), _w_spec((1, 1)),
        ],
        out_specs=[
            pl.BlockSpec((BE, D), lambda i: (i, 0)),
            pl.BlockSpec((BE, 1), lambda i: (i, 0)),
        ],
        out_shape=[
            jax.ShapeDtypeStruct((E, D), jnp.float32),
            jax.ShapeDtypeStruct((E, 1), jnp.float32),
        ],
    )(g1r, g2c, radial, edge_attr,
      we1r, We1e, We2, be2[None, :], Wc1, bc1[None, :],
      Wc2.T, Wa.T, ba[None, :])

    coord_update = coord_diff * cwn
    x_out = x.at[row].add(coord_update)
    m_i = jnp.zeros((N, D), jnp.float32).at[row].add(m)

    Wn1a = Wn1[:D]
    Wn1b = Wn1[D:]
    h_out = pl.pallas_call(
        _node_body,
        grid=(N // BN,),
        in_specs=[
            pl.BlockSpec((BN, D), lambda i: (i, 0)),
            pl.BlockSpec((BN, D), lambda i: (i, 0)),
            _w_spec((D, D)), _w_spec((D, D)), _w_spec((1, D)),
            _w_spec((D, D)), _w_spec((1, D)),
        ],
        out_specs=pl.BlockSpec((BN, D), lambda i: (i, 0)),
        out_shape=jax.ShapeDtypeStruct((N, D), jnp.float32),
    )(h, m_i, Wn1a, Wn1b, bn1[None, :], Wn2, bn2[None, :])

    return (h_out, x_out)
